# baseline (speedup 1.0000x reference)
import functools

import jax
import jax.numpy as jnp
from jax import lax
from jax.experimental import pallas as pl
from jax.experimental.pallas import tpu as pltpu
from jax.experimental.pallas import tpu_sc as plsc

_VOCAB = 1000000
_DIM = 64
_B = 16384
_NEG = 20

_NC = 2
_NS = 16
_NW = _NC * _NS
_BW = _B // _NW
_E = 32
_NCHUNK = _BW // _E
_NDOT = _NDOT_PER_E = 1 + _NEG
_CDOTS = _E * _NDOT_PER_E
_NGRP = _CDOTS // 16

_LN2 = 0.6931471805599453
_SQRT2 = 1.4142135623730951


def _neg_log(v):
    bits = lax.bitcast_convert_type(v, jnp.int32)
    e_raw = lax.shift_right_logical(bits, 23) & 0xFF
    m = lax.bitcast_convert_type((bits & 0x007FFFFF) | 0x3F800000, jnp.float32)
    big = m > _SQRT2
    m = jnp.where(big, m * 0.5, m)
    e_f = (e_raw - 127 + jnp.where(big, 1, 0)).astype(jnp.float32)
    z = (m - 1.0) / (m + 1.0)
    z2 = z * z
    p = z * (2.0 + z2 * (0.66666667 + z2 * (0.4 + z2 * (0.28571429 + z2 * 0.22222222))))
    return -(e_f * _LN2 + p)


def _worker_body(tgt_hbm, ctx_hbm, negf_hbm, inemb_hbm, outemb_hbm, out_hbm,
                 tidx, cidx, nidx, trows, crows, nrows, pbuf, accbuf, sem):
    wid = lax.axis_index("s") * _NC + lax.axis_index("c")
    base = wid * _BW
    lanes = lax.iota(jnp.int32, 16)

    def chunk_body(c, acc):
        ob = base + c * _E
        pltpu.sync_copy(tgt_hbm.at[pl.ds(ob, _E)], tidx)
        pltpu.sync_copy(ctx_hbm.at[pl.ds(ob, _E)], cidx)
        for j in range(5):
            pltpu.sync_copy(negf_hbm.at[pl.ds(ob * _NEG + j * 128, 128)], nidx.at[j])
        copies = [
            pltpu.async_copy(inemb_hbm.at[tidx], trows, sem),
            pltpu.async_copy(outemb_hbm.at[cidx], crows, sem),
        ]
        for j in range(5):
            copies.append(
                pltpu.async_copy(outemb_hbm.at[nidx.at[j]],
                                 nrows.at[pl.ds(j * 128, 128)], sem))
        for cp in copies:
            cp.wait()

        def dot_body(e, _):
            t = [trows[e, pl.ds(16 * j, 16)] for j in range(4)]
            d0 = e * _NDOT_PER_E
            accv = t[0] * crows[e, pl.ds(0, 16)]
            for j in range(1, 4):
                accv = accv + t[j] * crows[e, pl.ds(16 * j, 16)]
            pbuf[pl.ds(d0 * 16, 16)] = accv
            for n in range(_NEG):
                r = e * _NEG + n
                accv = t[0] * nrows[r, pl.ds(0, 16)]
                for j in range(1, 4):
                    accv = accv + t[j] * nrows[r, pl.ds(16 * j, 16)]
                pbuf[pl.ds((d0 + 1 + n) * 16, 16)] = accv
            return 0

        lax.fori_loop(0, _E, dot_body, 0)

        def grp_body(g, acc_in):
            gb = g * 16
            rowbase = (gb + lanes) * 16
            s = plsc.load_gather(pbuf, [rowbase])
            for j in range(1, 16):
                s = s + plsc.load_gather(pbuf, [rowbase + j])
            dotid = gb + lanes
            is_pos = lax.rem(dotid, jnp.int32(_NDOT_PER_E)) == 0
            x = jnp.where(is_pos, s, -s)
            sig = 1.0 / (1.0 + jnp.exp(-x))
            return acc_in + _neg_log(sig + 1e-7)

        return lax.fori_loop(0, _NGRP, grp_body, acc)

    acc = lax.fori_loop(0, _NCHUNK, chunk_body, jnp.zeros((16,), jnp.float32))
    accbuf[...] = acc
    pltpu.sync_copy(accbuf, out_hbm.at[wid])


@functools.partial(jax.jit, static_argnames=())
def _sg_loss(target, context, negflat, in_embed, out_embed):
    mesh = plsc.VectorSubcoreMesh(core_axis_name="c", subcore_axis_name="s")
    run = pl.kernel(
        _worker_body,
        out_type=jax.ShapeDtypeStruct((_NW, 16), jnp.float32),
        mesh=mesh,
        compiler_params=pltpu.CompilerParams(
            needs_layout_passes=False, use_tc_tiling_on_sc=False),
        scratch_types=[
            pltpu.VMEM((_E,), jnp.int32),
            pltpu.VMEM((_E,), jnp.int32),
            pltpu.VMEM((5, 128), jnp.int32),
            pltpu.VMEM((_E, _DIM), jnp.float32),
            pltpu.VMEM((_E, _DIM), jnp.float32),
            pltpu.VMEM((_E * _NEG, _DIM), jnp.float32),
            pltpu.VMEM((16 * _CDOTS,), jnp.float32),
            pltpu.VMEM((16,), jnp.float32),
            pltpu.SemaphoreType.DMA,
        ],
    )
    return run(target, context, negflat, in_embed, out_embed)


def kernel(target, context, neg_samples, in_embed, out_embed):
    negflat = neg_samples.reshape(-1)
    partials = _sg_loss(target, context, negflat, in_embed, out_embed)
    return jnp.sum(partials) / jnp.float32(_B)


# speedup vs baseline: 1.0624x; 1.0624x over previous
import functools

import jax
import jax.numpy as jnp
from jax import lax
from jax.experimental import pallas as pl
from jax.experimental.pallas import tpu as pltpu
from jax.experimental.pallas import tpu_sc as plsc

_VOCAB = 1000000
_DIM = 64
_B = 16384
_NEG = 20

_NC = 2
_NS = 16
_NW = _NC * _NS
_BW = _B // _NW
_E = 32
_NCHUNK = _BW // _E
_NDOT_PER_E = 1 + _NEG
_CDOTS = _E * _NDOT_PER_E
_NGRP = _CDOTS // 16

_LN2 = 0.6931471805599453
_SQRT2 = 1.4142135623730951


def _neg_log(v):
    bits = lax.bitcast_convert_type(v, jnp.int32)
    e_raw = lax.shift_right_logical(bits, 23) & 0xFF
    m = lax.bitcast_convert_type((bits & 0x007FFFFF) | 0x3F800000, jnp.float32)
    big = m > _SQRT2
    m = jnp.where(big, m * 0.5, m)
    e_f = (e_raw - 127 + jnp.where(big, 1, 0)).astype(jnp.float32)
    z = (m - 1.0) / (m + 1.0)
    z2 = z * z
    p = z * (2.0 + z2 * (0.66666667 + z2 * (0.4 + z2 * (0.28571429 + z2 * 0.22222222))))
    return -(e_f * _LN2 + p)


def _worker_body(tgt_hbm, ctx_hbm, negf_hbm, inemb_hbm, outemb_hbm, out_hbm,
                 tgtall, ctxall, negall, trows, crows, nrows, pbuf, accbuf,
                 sem0, sem1):
    wid = lax.axis_index("s") * _NC + lax.axis_index("c")
    base = wid * _BW
    lanes = lax.iota(jnp.int32, 16)
    sems = (sem0, sem1)

    pltpu.sync_copy(tgt_hbm.at[pl.ds(base, _BW)], tgtall)
    pltpu.sync_copy(ctx_hbm.at[pl.ds(base, _BW)], ctxall)
    pltpu.sync_copy(negf_hbm.at[pl.ds(base * _NEG, _BW * _NEG)], negall)

    def _fire(b, c):
        ob = pl.multiple_of(c * _E, _E)
        pltpu.async_copy(inemb_hbm.at[tgtall.at[pl.ds(ob, _E)]],
                         trows.at[b], sems[b])
        pltpu.async_copy(outemb_hbm.at[ctxall.at[pl.ds(ob, _E)]],
                         crows.at[b], sems[b])
        nb = pl.multiple_of(c * (_E * _NEG), _E * _NEG)
        for j in range(5):
            pltpu.async_copy(outemb_hbm.at[negall.at[pl.ds(nb + j * 128, 128)]],
                             nrows.at[b, pl.ds(j * 128, 128)], sems[b])

    def _drain(b):
        pltpu.make_async_copy(inemb_hbm.at[tgtall.at[pl.ds(0, _E)]],
                              trows.at[b], sems[b]).wait()
        pltpu.make_async_copy(outemb_hbm.at[ctxall.at[pl.ds(0, _E)]],
                              crows.at[b], sems[b]).wait()
        for j in range(5):
            pltpu.make_async_copy(outemb_hbm.at[negall.at[pl.ds(0, 128)]],
                                  nrows.at[b, pl.ds(j * 128, 128)],
                                  sems[b]).wait()

    def _compute(b, acc):
        def dot_body(e, _):
            t = [trows[b, e, pl.ds(16 * j, 16)] for j in range(4)]
            d0 = e * _NDOT_PER_E
            accv = t[0] * crows[b, e, pl.ds(0, 16)]
            for j in range(1, 4):
                accv = accv + t[j] * crows[b, e, pl.ds(16 * j, 16)]
            pbuf[pl.ds(d0 * 16, 16)] = accv
            for n in range(_NEG):
                r = e * _NEG + n
                accv = t[0] * nrows[b, r, pl.ds(0, 16)]
                for j in range(1, 4):
                    accv = accv + t[j] * nrows[b, r, pl.ds(16 * j, 16)]
                pbuf[pl.ds((d0 + 1 + n) * 16, 16)] = accv
            return 0

        lax.fori_loop(0, _E, dot_body, 0)

        def grp_body(g, acc_in):
            gb = g * 16
            rowbase = (gb + lanes) * 16
            s = plsc.load_gather(pbuf, [rowbase])
            for j in range(1, 16):
                s = s + plsc.load_gather(pbuf, [rowbase + j])
            dotid = gb + lanes
            is_pos = lax.rem(dotid, jnp.int32(_NDOT_PER_E)) == 0
            x = jnp.where(is_pos, s, -s)
            sig = 1.0 / (1.0 + jnp.exp(-x))
            return acc_in + _neg_log(sig + 1e-7)

        return lax.fori_loop(0, _NGRP, grp_body, acc)

    _fire(0, 0)

    def pair_body(h, acc):
        c0 = 2 * h
        _fire(1, c0 + 1)
        _drain(0)
        acc = _compute(0, acc)
        _fire(0, c0 + 2)
        _drain(1)
        return _compute(1, acc)

    acc = lax.fori_loop(0, _NCHUNK // 2 - 1, pair_body,
                        jnp.zeros((16,), jnp.float32))
    _fire(1, _NCHUNK - 1)
    _drain(0)
    acc = _compute(0, acc)
    _drain(1)
    acc = _compute(1, acc)

    accbuf[...] = acc
    pltpu.sync_copy(accbuf, out_hbm.at[wid])


@functools.partial(jax.jit, static_argnames=())
def _sg_loss(target, context, negflat, in_embed, out_embed):
    mesh = plsc.VectorSubcoreMesh(core_axis_name="c", subcore_axis_name="s")
    run = pl.kernel(
        _worker_body,
        out_type=jax.ShapeDtypeStruct((_NW, 16), jnp.float32),
        mesh=mesh,
        compiler_params=pltpu.CompilerParams(
            needs_layout_passes=False, use_tc_tiling_on_sc=False),
        scratch_types=[
            pltpu.VMEM((_BW,), jnp.int32),
            pltpu.VMEM((_BW,), jnp.int32),
            pltpu.VMEM((_BW * _NEG,), jnp.int32),
            pltpu.VMEM((2, _E, _DIM), jnp.float32),
            pltpu.VMEM((2, _E, _DIM), jnp.float32),
            pltpu.VMEM((2, _E * _NEG, _DIM), jnp.float32),
            pltpu.VMEM((16 * _CDOTS,), jnp.float32),
            pltpu.VMEM((16,), jnp.float32),
            pltpu.SemaphoreType.DMA,
            pltpu.SemaphoreType.DMA,
        ],
    )
    return run(target, context, negflat, in_embed, out_embed)


def kernel(target, context, neg_samples, in_embed, out_embed):
    negflat = neg_samples.reshape(-1)
    partials = _sg_loss(target, context, negflat, in_embed, out_embed)
    return jnp.sum(partials) / jnp.float32(_B)
